# baseline (speedup 1.0000x reference)
import jax
import jax.numpy as jnp
from jax.experimental import pallas as pl
from jax.experimental.pallas import tpu as pltpu

_B = 8
_C0 = 32
_TP0 = 258
_N = _C0 * _TP0
_L = 1024
_KSTRIPS = 4
_KTILE = _L // _KSTRIPS
_T_OUT = 1024


def _fused_kernel(x_ref, w_ref, b_ref, wb0_ref, bb0_ref, U0_ref,
                  wb1_ref, bb1_ref, U1_ref, o_ref, acc_ref, u1s_ref, sem_ref):
    j = pl.program_id(0)

    @pl.when(j == _KSTRIPS - 1)
    def _fetch_u1():
        pltpu.make_async_copy(U1_ref, u1s_ref, sem_ref).start()

    x_j = x_ref[:, pl.ds(j * _KTILE, _KTILE)]
    part = jnp.dot(x_j, w_ref[...], preferred_element_type=jnp.float32)

    @pl.when(j == 0)
    def _init():
        acc_ref[...] = part + b_ref[...]

    @pl.when(j != 0)
    def _accum():
        acc_ref[...] += part

    @pl.when(j == _KSTRIPS - 1)
    def _decode():
        h_full = acc_ref[...]
        hp = jnp.concatenate(
            [h_full[b:b + 1, c * _TP0:(c + 1) * _TP0]
             for b in range(_B) for c in range(_C0)],
            axis=0,
        )

        xcat = jnp.concatenate(
            [hp[:, 0:256], hp[:, 1:257], hp[:, 2:258]], axis=0)
        y = (jnp.dot(wb0_ref[...], xcat, preferred_element_type=jnp.float32)
             + bb0_ref[...])
        y = jax.nn.gelu(y, approximate=True)
        h = y + hp[:, 1:257]
        hp1 = jnp.dot(h, U0_ref[...],
                      preferred_element_type=jnp.float32)

        xcat1 = jnp.concatenate(
            [hp1[:, 0:512], hp1[:, 2:514], hp1[:, 4:516]], axis=0)
        y1 = (jnp.dot(wb1_ref[...], xcat1, preferred_element_type=jnp.float32)
              + bb1_ref[...])
        y1 = jax.nn.gelu(y1, approximate=True)
        h1 = y1 + hp1[:, 2:514]

        pltpu.make_async_copy(U1_ref, u1s_ref, sem_ref).wait()
        o_ref[...] = jnp.dot(h1, u1s_ref[...],
                             preferred_element_type=jnp.float32)


def kernel(x, lin_wT, lin_b, wb_0, bb_0, U_0, wb_1, bb_1, U_1):
    out2d = pl.pallas_call(
        _fused_kernel,
        out_shape=jax.ShapeDtypeStruct((_B * _C0, _T_OUT), jnp.float32),
        grid=(_KSTRIPS,),
        in_specs=[
            pl.BlockSpec((_B, _L), lambda j: (0, 0)),
            pl.BlockSpec((_KTILE, _N), lambda j: (j, 0)),
            pl.BlockSpec((1, _N), lambda j: (0, 0)),
            pl.BlockSpec(wb_0.shape, lambda j: (0, 0)),
            pl.BlockSpec(bb_0.shape, lambda j: (0, 0)),
            pl.BlockSpec(U_0.shape, lambda j: (0, 0)),
            pl.BlockSpec(wb_1.shape, lambda j: (0, 0)),
            pl.BlockSpec(bb_1.shape, lambda j: (0, 0)),
            pl.BlockSpec(memory_space=pl.ANY),
        ],
        out_specs=pl.BlockSpec((_B * _C0, _T_OUT), lambda j: (0, 0)),
        scratch_shapes=[pltpu.VMEM((_B, _N), jnp.float32),
                        pltpu.VMEM((512, _T_OUT), jnp.float32),
                        pltpu.SemaphoreType.DMA],
        compiler_params=pltpu.CompilerParams(
            dimension_semantics=("arbitrary",)),
    )(x, lin_wT, lin_b, wb_0, bb_0, U_0, wb_1, bb_1, U_1)

    return out2d.reshape(_B, _C0, _T_OUT)


# speedup vs baseline: 1.0073x; 1.0073x over previous
import jax
import jax.numpy as jnp
from jax.experimental import pallas as pl
from jax.experimental.pallas import tpu as pltpu

_B = 8
_C0 = 32
_TP0 = 258
_N = _C0 * _TP0
_L = 1024
_KSTRIPS = 4
_KTILE = _L // _KSTRIPS
_T_OUT = 1024


def _fused_kernel(x_ref, w_ref, b_ref, wb0_ref, bb0_ref, U0_ref,
                  wb1_ref, bb1_ref, U1_ref, o_ref, acc_ref, u1s_ref, sem_ref,
                  obufL_ref, obufR_ref, osem_ref):
    j = pl.program_id(0)

    @pl.when(j == _KSTRIPS - 1)
    def _fetch_u1():
        pltpu.make_async_copy(U1_ref, u1s_ref, sem_ref).start()

    x_j = x_ref[:, pl.ds(j * _KTILE, _KTILE)]
    part = jnp.dot(x_j, w_ref[...], preferred_element_type=jnp.float32)

    @pl.when(j == 0)
    def _init():
        acc_ref[...] = part + b_ref[...]

    @pl.when(j != 0)
    def _accum():
        acc_ref[...] += part

    @pl.when(j == _KSTRIPS - 1)
    def _decode():
        h_full = acc_ref[...]
        hp = jnp.concatenate(
            [h_full[b:b + 1, c * _TP0:(c + 1) * _TP0]
             for b in range(_B) for c in range(_C0)],
            axis=0,
        )

        xcat = jnp.concatenate(
            [hp[:, 0:256], hp[:, 1:257], hp[:, 2:258]], axis=0)
        y = (jnp.dot(wb0_ref[...], xcat, preferred_element_type=jnp.float32)
             + bb0_ref[...])
        y = jax.nn.gelu(y, approximate=True)
        h = y + hp[:, 1:257]
        hp1 = jnp.dot(h, U0_ref[...],
                      preferred_element_type=jnp.float32)

        xcat1 = jnp.concatenate(
            [hp1[:, 0:512], hp1[:, 2:514], hp1[:, 4:516]], axis=0)
        y1 = (jnp.dot(wb1_ref[...], xcat1, preferred_element_type=jnp.float32)
              + bb1_ref[...])
        y1 = jax.nn.gelu(y1, approximate=True)
        h1 = y1 + hp1[:, 2:514]

        pltpu.make_async_copy(U1_ref, u1s_ref, sem_ref).wait()
        obufL_ref[...] = jnp.dot(h1, u1s_ref[:, 0:512],
                                 preferred_element_type=jnp.float32)
        cpL = pltpu.make_async_copy(obufL_ref, o_ref.at[:, 0:512],
                                    osem_ref.at[0])
        cpL.start()
        obufR_ref[...] = jnp.dot(h1, u1s_ref[:, 512:1024],
                                 preferred_element_type=jnp.float32)
        cpR = pltpu.make_async_copy(obufR_ref, o_ref.at[:, 512:1024],
                                    osem_ref.at[1])
        cpR.start()
        cpL.wait()
        cpR.wait()


def kernel(x, lin_wT, lin_b, wb_0, bb_0, U_0, wb_1, bb_1, U_1):
    out2d = pl.pallas_call(
        _fused_kernel,
        out_shape=jax.ShapeDtypeStruct((_B * _C0, _T_OUT), jnp.float32),
        grid=(_KSTRIPS,),
        in_specs=[
            pl.BlockSpec((_B, _L), lambda j: (0, 0)),
            pl.BlockSpec((_KTILE, _N), lambda j: (j, 0)),
            pl.BlockSpec((1, _N), lambda j: (0, 0)),
            pl.BlockSpec(wb_0.shape, lambda j: (0, 0)),
            pl.BlockSpec(bb_0.shape, lambda j: (0, 0)),
            pl.BlockSpec(U_0.shape, lambda j: (0, 0)),
            pl.BlockSpec(wb_1.shape, lambda j: (0, 0)),
            pl.BlockSpec(bb_1.shape, lambda j: (0, 0)),
            pl.BlockSpec(memory_space=pl.ANY),
        ],
        out_specs=pl.BlockSpec(memory_space=pl.ANY),
        scratch_shapes=[pltpu.VMEM((_B, _N), jnp.float32),
                        pltpu.VMEM((512, _T_OUT), jnp.float32),
                        pltpu.SemaphoreType.DMA,
                        pltpu.VMEM((_B * _C0, 512), jnp.float32),
                        pltpu.VMEM((_B * _C0, 512), jnp.float32),
                        pltpu.SemaphoreType.DMA((2,))],
        compiler_params=pltpu.CompilerParams(
            dimension_semantics=("arbitrary",)),
    )(x, lin_wT, lin_b, wb_0, bb_0, U_0, wb_1, bb_1, U_1)

    return out2d.reshape(_B, _C0, _T_OUT)
